# baseline (speedup 1.0000x reference)
import functools

import jax
import jax.numpy as jnp
from jax import lax
from jax.experimental import pallas as pl
from jax.experimental.pallas import tpu as pltpu
from jax.experimental.pallas import tpu_sc as plsc

B = 16384
K = 32
NC = 2
NS = 16
NW = NC * NS
BPW = B // NW
CH = 128
NCHUNK = BPW // CH
RB = 16
NBLK = BPW // RB

_mesh = plsc.VectorSubcoreMesh(core_axis_name="c", subcore_axis_name="s")


@functools.partial(
    pl.kernel,
    mesh=_mesh,
    out_type=jax.ShapeDtypeStruct((B,), jnp.float32),
    compiler_params=pltpu.CompilerParams(
        needs_layout_passes=False, use_tc_tiling_on_sc=False),
    scratch_types=[
        pltpu.VMEM((BPW,), jnp.int32),
        pltpu.VMEM((BPW,), jnp.int32),
        pltpu.VMEM((BPW, K), jnp.float32),
        pltpu.VMEM((BPW, K), jnp.float32),
        pltpu.VMEM((BPW,), jnp.float32),
        pltpu.SemaphoreType.DMA,
    ],
)
def _sc_kernel(uidx_hbm, vidx_hbm, uemb_hbm, vemb_hbm, out_hbm,
               uidx, vidx, urows, vrows, outv, sem):
    wid = lax.axis_index("s") * NC + lax.axis_index("c")
    base = wid * BPW
    pltpu.sync_copy(uidx_hbm.at[pl.ds(base, BPW)], uidx)
    pltpu.sync_copy(vidx_hbm.at[pl.ds(base, BPW)], vidx)
    copies = []
    for c in range(NCHUNK):
        copies.append(pltpu.async_copy(
            uemb_hbm.at[uidx.at[pl.ds(c * CH, CH)]],
            urows.at[pl.ds(c * CH, CH)], sem))
        copies.append(pltpu.async_copy(
            vemb_hbm.at[vidx.at[pl.ds(c * CH, CH)]],
            vrows.at[pl.ds(c * CH, CH)], sem))
    for cp in copies:
        cp.wait()

    lanes = lax.iota(jnp.int32, RB)

    def block(bi, _):
        rid = bi * RB + lanes
        acc = jnp.zeros((RB,), jnp.float32)
        for k in range(K):
            cid = jnp.full((RB,), k, jnp.int32)
            u = plsc.load_gather(urows, [rid, cid])
            v = plsc.load_gather(vrows, [rid, cid])
            acc = acc + u * v
        outv[pl.ds(bi * RB, RB)] = acc
        return 0

    lax.fori_loop(0, NBLK, block, 0)
    pltpu.sync_copy(outv, out_hbm.at[pl.ds(base, BPW)])


def kernel(user_index, item_index, Uemb, Vemb):
    return _sc_kernel(user_index.astype(jnp.int32), item_index.astype(jnp.int32),
                      Uemb, Vemb)
